# baseline (speedup 1.0000x reference)
import math

import jax
import jax.numpy as jnp
from jax import lax
from jax.experimental import pallas as pl
from jax.experimental.pallas import tpu as pltpu
from jax.experimental.pallas import tpu_sc as plsc

EMB = 64
SCALE = math.sqrt(EMB)
B_TOK = 4096 * 50
HALF_V = 512000
NC, NS, L = 2, 16, 16
NW = NC * NS
N_PER_W = B_TOK // NW
G = 128
IDX_ROWS = N_PER_W // G
K = 5
CHUNK = K * G
NCH = N_PER_W // CHUNK
FC = 4096
BB = 128


def _fmt_body(in_l, in_r, out_ref):
    eye = jnp.eye(EMB, dtype=jnp.float32)
    dn = (((0,), (0,)), ((), ()))
    out_ref[:, :EMB] = lax.dot_general(
        in_l[...], eye, dn, preferred_element_type=jnp.float32
    )
    out_ref[:, EMB:] = lax.dot_general(
        in_r[...], eye, dn, preferred_element_type=jnp.float32
    )


@jax.jit
def _fmt_call(table_t):
    return pl.pallas_call(
        _fmt_body,
        grid=(HALF_V // FC,),
        in_specs=[
            pl.BlockSpec((EMB, FC), lambda i: (0, i)),
            pl.BlockSpec(
                (EMB, FC),
                lambda i: (0, jnp.minimum(i + HALF_V // FC, 1000000 // FC)),
            ),
        ],
        out_specs=pl.BlockSpec((FC, 2 * EMB), lambda i: (i, 0)),
        out_shape=jax.ShapeDtypeStruct((HALF_V, 2 * EMB), jnp.float32),
    )(table_t, table_t)


def _gather_body(tok_hbm, tbl_hbm, out_hbm, idx_v, buf, gsem):
    wid = lax.axis_index("s") * NC + lax.axis_index("c")
    base = wid * N_PER_W
    pltpu.sync_copy(tok_hbm.at[wid], idx_v)

    def chunk_body(g, carry):
        cps = [
            pltpu.async_copy(
                tbl_hbm.at[idx_v.at[g * K + j]],
                buf.at[pl.ds(j * G, G)],
                gsem,
            )
            for j in range(K)
        ]
        for cp in cps:
            cp.wait()
        pltpu.sync_copy(buf, out_hbm.at[pl.ds(base + g * CHUNK, CHUNK)])
        return carry

    lax.fori_loop(0, NCH, chunk_body, 0)


@jax.jit
def _emb_call(tok3, tbl2):
    mesh = plsc.VectorSubcoreMesh(core_axis_name="c", subcore_axis_name="s")
    return pl.kernel(
        _gather_body,
        mesh=mesh,
        compiler_params=pltpu.CompilerParams(use_tc_tiling_on_sc=False),
        out_type=jax.ShapeDtypeStruct((B_TOK, 2 * EMB), jnp.float32),
        scratch_types=[
            pltpu.VMEM((IDX_ROWS, G), jnp.int32),
            pltpu.VMEM((CHUNK, 2 * EMB), jnp.float32),
            pltpu.SemaphoreType.DMA,
        ],
    )(tok3, tbl2)


def _finish_body(rows_ref, par_ref, out_ref):
    rows = rows_ref[...]
    a = rows[:, :EMB].reshape(BB, 50, EMB)
    b = rows[:, EMB:].reshape(BB, 50, EMB)
    p = par_ref[...].reshape(BB, 50, 1)
    sel = jnp.where(p == 0, a, b) * SCALE
    out_ref[...] = jnp.transpose(sel, (1, 2, 0))


@jax.jit
def _finish_call(rows, par):
    return pl.pallas_call(
        _finish_body,
        grid=(4096 // BB,),
        in_specs=[
            pl.BlockSpec((BB * 50, 2 * EMB), lambda i: (i, 0)),
            pl.BlockSpec((BB, 50), lambda i: (i, 0)),
        ],
        out_specs=pl.BlockSpec((50, EMB, BB), lambda i: (0, 0, i)),
        out_shape=jax.ShapeDtypeStruct((50, EMB, 4096), jnp.float32),
    )(rows, par)


def kernel(tokens, table):
    tok = tokens.astype(jnp.int32)
    par = (tok >= HALF_V).astype(jnp.int32)
    tok3 = (tok - par * HALF_V).reshape(NW, IDX_ROWS, G)
    tbl2 = _fmt_call(table.T)
    rows = _emb_call(tok3, tbl2)
    return jnp.transpose(_finish_call(rows, par), (2, 0, 1))


# speedup vs baseline: 1.0586x; 1.0586x over previous
import math

import jax
import jax.numpy as jnp
from jax import lax
from jax.experimental import pallas as pl
from jax.experimental.pallas import tpu as pltpu
from jax.experimental.pallas import tpu_sc as plsc

EMB = 64
SCALE = math.sqrt(EMB)
B_TOK = 4096 * 50
HALF_V = 524288
NC, NS, L = 2, 16, 16
NW = NC * NS
N_PER_W = B_TOK // NW
G = 128
IDX_ROWS = N_PER_W // G
K = 5
CHUNK = K * G
NCH = N_PER_W // CHUNK
FC = 8192
BB = 256


def _fmt_body(in_l, in_r, out_ref):
    out_ref[:, :EMB] = jnp.transpose(in_l[...])
    out_ref[:, EMB:] = jnp.transpose(in_r[...])


@jax.jit
def _fmt_call(table_t):
    return pl.pallas_call(
        _fmt_body,
        grid=(HALF_V // FC,),
        in_specs=[
            pl.BlockSpec((EMB, FC), lambda i: (0, i)),
            pl.BlockSpec(
                (EMB, FC),
                lambda i: (0, jnp.minimum(i + HALF_V // FC, 1000000 // FC)),
            ),
        ],
        out_specs=pl.BlockSpec((FC, 2 * EMB), lambda i: (i, 0)),
        out_shape=jax.ShapeDtypeStruct((HALF_V, 2 * EMB), jnp.float32),
    )(table_t, table_t)


def _gather_body(tok_hbm, tbl_hbm, out_hbm, idx_v, buf, gsem):
    wid = lax.axis_index("s") * NC + lax.axis_index("c")
    base = wid * N_PER_W
    pltpu.sync_copy(tok_hbm.at[wid], idx_v)

    def chunk_body(g, carry):
        cps = [
            pltpu.async_copy(
                tbl_hbm.at[idx_v.at[g * K + j]],
                buf.at[pl.ds(j * G, G)],
                gsem,
            )
            for j in range(K)
        ]
        for cp in cps:
            cp.wait()
        pltpu.sync_copy(buf, out_hbm.at[pl.ds(base + g * CHUNK, CHUNK)])
        return carry

    lax.fori_loop(0, NCH, chunk_body, 0)


@jax.jit
def _emb_call(tok3, tbl2):
    mesh = plsc.VectorSubcoreMesh(core_axis_name="c", subcore_axis_name="s")
    return pl.kernel(
        _gather_body,
        mesh=mesh,
        compiler_params=pltpu.CompilerParams(use_tc_tiling_on_sc=False),
        out_type=jax.ShapeDtypeStruct((B_TOK, 2 * EMB), jnp.float32),
        scratch_types=[
            pltpu.VMEM((IDX_ROWS, G), jnp.int32),
            pltpu.VMEM((CHUNK, 2 * EMB), jnp.float32),
            pltpu.SemaphoreType.DMA,
        ],
    )(tok3, tbl2)


def _finish_body(rows_ref, par_ref, out_ref):
    rows = rows_ref[...]
    a = rows[:, :EMB].reshape(BB, 50, EMB)
    b = rows[:, EMB:].reshape(BB, 50, EMB)
    p = par_ref[...].reshape(BB, 50, 1)
    sel = jnp.where(p == 0, a, b) * SCALE
    out_ref[...] = jnp.transpose(sel, (1, 2, 0))


@jax.jit
def _finish_call(rows, par):
    return pl.pallas_call(
        _finish_body,
        grid=(4096 // BB,),
        in_specs=[
            pl.BlockSpec((BB * 50, 2 * EMB), lambda i: (i, 0)),
            pl.BlockSpec((BB, 50), lambda i: (i, 0)),
        ],
        out_specs=pl.BlockSpec((50, EMB, BB), lambda i: (0, 0, i)),
        out_shape=jax.ShapeDtypeStruct((50, EMB, 4096), jnp.float32),
    )(rows, par)


def kernel(tokens, table):
    tok = tokens.astype(jnp.int32)
    par = (tok >= HALF_V).astype(jnp.int32)
    tok3 = (tok - par * HALF_V).reshape(NW, IDX_ROWS, G)
    tbl2 = _fmt_call(table.T)
    rows = _emb_call(tok3, tbl2)
    return jnp.transpose(_finish_call(rows, par), (2, 0, 1))
